# baseline (speedup 1.0000x reference)
import jax
import jax.numpy as jnp
import numpy as np
from jax.experimental import pallas as pl
from jax.experimental.pallas import tpu as pltpu

_N = 16384
_D = 768
_T = 2048
_B = 4
_DECAY = 0.9
_ROUNDS = 10
_THRESH = 0.5

_NB = 8
_BN = _N // _NB
_TB = 8
_BT = _T // _TB
_SCALE = np.float32(1.0 / np.sqrt(np.float32(_D)))


def _pool_body(qs_ref, out_ref):
    j = pl.program_id(1)

    @pl.when(j == 0)
    def _():
        out_ref[...] = jnp.zeros_like(out_ref)

    out_ref[...] += jnp.sum(qs_ref[...], axis=1, keepdims=False)[None]


def _scores_body(qpart_ref, wq_ref, bq_ref, keys_ref, s_ref, qv):
    j = pl.program_id(1)

    @pl.when(j == 0)
    def _():
        q = jnp.sum(qpart_ref[...], axis=0) * (1.0 / _T)
        qv[...] = jax.lax.dot_general(
            q, wq_ref[...], (((1,), (1,)), ((), ())),
            preferred_element_type=jnp.float32) + bq_ref[...]

    s_ref[...] = (jax.lax.dot_general(
        qv[...], keys_ref[...], (((1,), (1,)), ((), ())),
        preferred_element_type=jnp.float32) * _SCALE)[None]


def _retrieve_body(s_ref, pri_ref, ages_ref, vm_ref, values_ref, out_ref,
                   w_scr, acc):
    c = pl.program_id(0)
    j = pl.program_id(1)

    @pl.when(j == 0)
    def _rounds():
        s = s_ref[...]
        log_decay = np.float32(np.log(_DECAY))
        eff0 = pri_ref[...] * jnp.exp(ages_ref[...] * log_decay)
        vm = vm_ref[...]
        wsum = jnp.zeros_like(s)
        for r in range(_ROUNDS):
            eff = eff0 * np.float32(_DECAY ** r)
            logits = s + jnp.log(eff + 1e-8)
            m = jnp.max(logits, axis=(0, 2), keepdims=True)
            pex = jnp.exp(logits - m)
            den = jnp.sum(pex, axis=(0, 2), keepdims=True)
            active = jax.nn.sigmoid((eff - _THRESH) * 10.0) * vm
            num = pex * active
            wsum += num / (jnp.sum(num, axis=(0, 2), keepdims=True)
                           + 1e-8 * den)
        w_scr[...] = wsum
        acc[...] = jnp.zeros_like(acc)

    nb_half = _NB // 2
    acc[...] += jax.lax.dot_general(
        w_scr[c * nb_half + j], values_ref[...], (((1,), (0,)), ((), ())),
        preferred_element_type=jnp.float32)

    @pl.when(j == nb_half - 1)
    def _():
        out_ref[...] = acc[...][None]


def _combine_body(part_ref, wc_ref, bc_ref, out_ref):
    p = jnp.sum(part_ref[...], axis=0)
    out_ref[...] = jax.lax.dot_general(
        p, wc_ref[...], (((1,), (1,)), ((), ())),
        preferred_element_type=jnp.float32) * (1.0 / _ROUNDS) + bc_ref[...]


@jax.jit
def kernel(query_states, keys, values, priorities, Wq, bq, Wc, bc, ages,
           valid_mask):
    B, T, D = query_states.shape

    pri = priorities.reshape(_NB, 1, _BN)
    ages_f = ages.astype(jnp.float32).reshape(_NB, 1, _BN)
    vm = valid_mask.astype(jnp.float32).reshape(_NB, 1, _BN)
    bq2 = bq.reshape(1, D)
    bc2 = bc.reshape(1, D)

    par = pltpu.CompilerParams(
        dimension_semantics=("parallel", "arbitrary"))

    qpart = pl.pallas_call(
        _pool_body,
        grid=(2, _TB // 2),
        in_specs=[pl.BlockSpec((B, _BT, D),
                               lambda c, j: (0, c * (_TB // 2) + j, 0))],
        out_specs=pl.BlockSpec((1, B, D), lambda c, j: (c, 0, 0)),
        out_shape=jax.ShapeDtypeStruct((2, B, D), jnp.float32),
        compiler_params=par,
    )(query_states)

    s = pl.pallas_call(
        _scores_body,
        grid=(2, _NB // 2),
        in_specs=[
            pl.BlockSpec((2, B, D), lambda c, j: (0, 0, 0)),
            pl.BlockSpec((D, D), lambda c, j: (0, 0)),
            pl.BlockSpec((1, D), lambda c, j: (0, 0)),
            pl.BlockSpec((_BN, D), lambda c, j: (c * (_NB // 2) + j, 0)),
        ],
        out_specs=pl.BlockSpec((1, B, _BN),
                               lambda c, j: (c * (_NB // 2) + j, 0, 0)),
        out_shape=jax.ShapeDtypeStruct((_NB, B, _BN), jnp.float32),
        scratch_shapes=[pltpu.VMEM((B, D), jnp.float32)],
        compiler_params=par,
    )(qpart, Wq, bq2, keys)

    partial = pl.pallas_call(
        _retrieve_body,
        grid=(2, _NB // 2),
        in_specs=[
            pl.BlockSpec((_NB, B, _BN), lambda c, j: (0, 0, 0)),
            pl.BlockSpec((_NB, 1, _BN), lambda c, j: (0, 0, 0)),
            pl.BlockSpec((_NB, 1, _BN), lambda c, j: (0, 0, 0)),
            pl.BlockSpec((_NB, 1, _BN), lambda c, j: (0, 0, 0)),
            pl.BlockSpec((_BN, D), lambda c, j: (c * (_NB // 2) + j, 0)),
        ],
        out_specs=pl.BlockSpec((1, B, D), lambda c, j: (c, 0, 0)),
        out_shape=jax.ShapeDtypeStruct((2, B, D), jnp.float32),
        scratch_shapes=[
            pltpu.VMEM((_NB, B, _BN), jnp.float32),
            pltpu.VMEM((B, D), jnp.float32),
        ],
        compiler_params=par,
    )(s, pri, ages_f, vm, values)

    out = pl.pallas_call(
        _combine_body,
        in_specs=[
            pl.BlockSpec((2, B, D), lambda: (0, 0, 0)),
            pl.BlockSpec((D, D), lambda: (0, 0)),
            pl.BlockSpec((1, D), lambda: (0, 0)),
        ],
        out_specs=pl.BlockSpec((B, D), lambda: (0, 0)),
        out_shape=jax.ShapeDtypeStruct((B, D), jnp.float32),
    )(partial, Wc, bc2)
    return out


# speedup vs baseline: 1.0817x; 1.0817x over previous
import jax
import jax.numpy as jnp
import numpy as np
from jax.experimental import pallas as pl
from jax.experimental.pallas import tpu as pltpu

_N = 16384
_D = 768
_T = 2048
_B = 4
_DECAY = 0.9
_ROUNDS = 10
_THRESH = 0.5

_NB = 16
_BN = _N // _NB
_NH = _NB // 2
_TB = 16
_BT = _T // _TB
_SCALE = np.float32(1.0 / np.sqrt(np.float32(_D)))


def _body(qs_lo, qs_hi, keys_lo, keys_hi, val_lo, val_hi,
          pri_ref, ages_ref, vm_ref, wq_ref, bq_ref, wc_ref, bc_ref,
          out_ref, qvec, s_scr, w_scr, acc):
    p = pl.program_id(0)
    j = pl.program_id(1)
    f32 = jnp.float32

    @pl.when(jnp.logical_and(p == 0, j == 0))
    def _init():
        qvec[...] = jnp.zeros_like(qvec)
        acc[...] = jnp.zeros_like(acc)

    @pl.when(p == 0)
    def _pool():
        qvec[...] += (jnp.sum(qs_lo[...], axis=1)
                      + jnp.sum(qs_hi[...], axis=1))

    @pl.when(jnp.logical_and(p == 1, j == 0))
    def _project_q():
        q = qvec[...] * (1.0 / _T)
        qvec[...] = jax.lax.dot_general(
            q, wq_ref[...], (((1,), (1,)), ((), ())),
            preferred_element_type=f32) + bq_ref[...]

    @pl.when(p == 1)
    def _scores():
        s_scr[j] = jax.lax.dot_general(
            qvec[...], keys_lo[...], (((1,), (1,)), ((), ())),
            preferred_element_type=f32) * _SCALE
        s_scr[j + _NH] = jax.lax.dot_general(
            qvec[...], keys_hi[...], (((1,), (1,)), ((), ())),
            preferred_element_type=f32) * _SCALE

    @pl.when(jnp.logical_and(p == 2, j == 0))
    def _rounds():
        s = s_scr[...]
        log_decay = np.float32(np.log(_DECAY))
        eff0 = pri_ref[...] * jnp.exp(ages_ref[...] * log_decay)
        vm = vm_ref[...]
        wsum = jnp.zeros_like(s)
        for r in range(_ROUNDS):
            eff = eff0 * np.float32(_DECAY ** r)
            logits = s + jnp.log(eff + 1e-8)
            m = jnp.max(logits, axis=(0, 2), keepdims=True)
            pex = jnp.exp(logits - m)
            den = jnp.sum(pex, axis=(0, 2), keepdims=True)
            active = jax.nn.sigmoid((eff - _THRESH) * 10.0) * vm
            num = pex * active
            wsum += num / (jnp.sum(num, axis=(0, 2), keepdims=True)
                           + 1e-8 * den)
        w_scr[...] = wsum

    @pl.when(p == 2)
    def _retrieve():
        acc[...] += (jax.lax.dot_general(
            w_scr[j], val_lo[...], (((1,), (0,)), ((), ())),
            preferred_element_type=f32)
            + jax.lax.dot_general(
                w_scr[j + _NH], val_hi[...], (((1,), (0,)), ((), ())),
                preferred_element_type=f32))

    @pl.when(jnp.logical_and(p == 2, j == _NH - 1))
    def _project_out():
        out_ref[...] = jax.lax.dot_general(
            acc[...], wc_ref[...], (((1,), (1,)), ((), ())),
            preferred_element_type=f32) * (1.0 / _ROUNDS) + bc_ref[...]


@jax.jit
def kernel(query_states, keys, values, priorities, Wq, bq, Wc, bc, ages,
           valid_mask):
    B, T, D = query_states.shape

    pri = priorities.reshape(_NB, 1, _BN)
    ages_f = ages.astype(jnp.float32).reshape(_NB, 1, _BN)
    vm = valid_mask.astype(jnp.float32).reshape(_NB, 1, _BN)
    bq2 = bq.reshape(1, D)
    bc2 = bc.reshape(1, D)

    th = _TB // 2
    qs_lo_map = lambda p, j: (0, jnp.where(p == 0, j, th - 1), 0)
    qs_hi_map = lambda p, j: (0, jnp.where(p == 0, j + th, _TB - 1), 0)
    k_lo_map = lambda p, j: (jnp.where(p == 1, j, jnp.where(p == 0, 0, _NH - 1)), 0)
    k_hi_map = lambda p, j: (jnp.where(p == 1, j + _NH,
                                       jnp.where(p == 0, _NH, _NB - 1)), 0)
    v_lo_map = lambda p, j: (jnp.where(p == 2, j, 0), 0)
    v_hi_map = lambda p, j: (jnp.where(p == 2, j + _NH, _NH), 0)

    out = pl.pallas_call(
        _body,
        grid=(3, _NH),
        in_specs=[
            pl.BlockSpec((B, _BT, D), qs_lo_map),
            pl.BlockSpec((B, _BT, D), qs_hi_map),
            pl.BlockSpec((_BN, D), k_lo_map),
            pl.BlockSpec((_BN, D), k_hi_map),
            pl.BlockSpec((_BN, D), v_lo_map),
            pl.BlockSpec((_BN, D), v_hi_map),
            pl.BlockSpec((_NB, 1, _BN), lambda p, j: (0, 0, 0)),
            pl.BlockSpec((_NB, 1, _BN), lambda p, j: (0, 0, 0)),
            pl.BlockSpec((_NB, 1, _BN), lambda p, j: (0, 0, 0)),
            pl.BlockSpec((_D, _D), lambda p, j: (0, 0)),
            pl.BlockSpec((1, _D), lambda p, j: (0, 0)),
            pl.BlockSpec((_D, _D), lambda p, j: (0, 0)),
            pl.BlockSpec((1, _D), lambda p, j: (0, 0)),
        ],
        out_specs=pl.BlockSpec((B, D), lambda p, j: (0, 0)),
        out_shape=jax.ShapeDtypeStruct((B, D), jnp.float32),
        scratch_shapes=[
            pltpu.VMEM((B, D), jnp.float32),
            pltpu.VMEM((_NB, B, _BN), jnp.float32),
            pltpu.VMEM((_NB, B, _BN), jnp.float32),
            pltpu.VMEM((B, D), jnp.float32),
        ],
    )(query_states, query_states, keys, keys, values, values,
      pri, ages_f, vm, Wq, bq2, Wc, bc2)
    return out
